# Initial kernel scaffold; baseline (speedup 1.0000x reference)
import jax
import jax.numpy as jnp
from jax.experimental import pallas as pl


def kernel(input, beam_id, W):
    raise NotImplementedError("write your pallas kernel here")


# speedup vs baseline: 1.6388x; 1.6388x over previous
import jax
import jax.numpy as jnp
from jax import lax
from jax.experimental import pallas as pl
from jax.experimental.pallas import tpu as pltpu
from jax.experimental.pallas import tpu_sc as plsc

B = 8192
F = 257
C = 8
N = 4
NBEAMS = 16
ROW_X = 2 * F * C
ROW_O = 2 * F * N
ROW_W = N * 2 * F * C
G = 16
HALF = B // 2
NFT = (F + 15) // 16


def _compute_chunk(x_v, w_v, o_v):
    iota = lax.iota(jnp.int32, 16)
    i8 = iota * 8
    i4 = iota * 4

    def f_body(ft, carry):
        f0 = jnp.minimum(ft * 16, F - 16)
        for npair in range(2):
            n0 = 2 * npair
            wr = [[w_v[pl.ds((c * N + n0 + k) * F + f0, 16)] for k in range(2)]
                  for c in range(C)]
            wi = [[w_v[pl.ds(((C + c) * N + n0 + k) * F + f0, 16)] for k in range(2)]
                  for c in range(C)]
            for g in range(G):
                xrow = x_v.at[g]
                orow = o_v.at[g]
                ar0 = jnp.zeros((16,), jnp.float32)
                ar1 = jnp.zeros((16,), jnp.float32)
                ai0 = jnp.zeros((16,), jnp.float32)
                ai1 = jnp.zeros((16,), jnp.float32)
                for c in range(C):
                    colr = i8 + (f0 * C + c)
                    xr = plsc.load_gather(xrow, [colr])
                    xi = plsc.load_gather(xrow, [colr + F * C])
                    ar0 = ar0 + xr * wr[c][0] + xi * wi[c][0]
                    ar1 = ar1 + xr * wr[c][1] + xi * wi[c][1]
                    ai0 = ai0 + xi * wr[c][0] - xr * wi[c][0]
                    ai1 = ai1 + xi * wr[c][1] - xr * wi[c][1]
                ob = i4 + f0 * N
                plsc.store_scatter(orow, [ob + n0], ar0)
                plsc.store_scatter(orow, [ob + (n0 + 1)], ar1)
                plsc.store_scatter(orow, [ob + (F * N + n0)], ai0)
                plsc.store_scatter(orow, [ob + (F * N + n0 + 1)], ai1)
        return carry

    lax.fori_loop(0, NFT, f_body, 0)


def _sc_body(x_hbm, bid_hbm, w_hbm, out_hbm,
             bid_v, idx_v, w_v, x_v, o_v, sem_g, sem_s):
    core = lax.axis_index("c")
    beam = lax.axis_index("s")
    half_base = core * HALF

    pltpu.sync_copy(bid_hbm.at[pl.ds(half_base, HALF)], bid_v)
    pltpu.sync_copy(w_hbm.at[beam], w_v)

    iota = lax.iota(jnp.int32, 16)

    def comp_body(i, cursor):
        bid = bid_v[pl.ds(i * 16, 16)]
        mask = bid == beam
        vals = iota + (half_base + i * 16)
        mask_i32 = jnp.where(mask, jnp.int32(1), jnp.int32(0))
        incl = plsc.cumsum(mask_i32)
        pos = cursor + incl - mask_i32
        plsc.store_scatter(idx_v, [pos], vals, mask=mask)
        return cursor + jnp.sum(mask_i32)

    n = lax.fori_loop(0, HALF // 16, comp_body, jnp.int32(0))

    @pl.when(n > 0)
    def _():
        pad = plsc.load_gather(idx_v, [jnp.full((16,), n - 1, jnp.int32)])
        idx_v[pl.ds(n, 16)] = pad
        nchunks = (n + G - 1) // G

        def chunk_body(j, carry):
            ivec = idx_v[pl.ds(j * G, G)]
            pltpu.async_copy(x_hbm.at[ivec], x_v, sem_g).wait()
            _compute_chunk(x_v, w_v, o_v)
            pltpu.async_copy(o_v, out_hbm.at[ivec], sem_s).wait()
            return carry

        lax.fori_loop(0, nchunks, chunk_body, 0)


def _beamform_sc(x_flat, bid, w_flat):
    mesh = plsc.VectorSubcoreMesh(
        core_axis_name="c", subcore_axis_name="s",
        num_cores=2, num_subcores=16)
    return pl.kernel(
        _sc_body,
        out_type=jax.ShapeDtypeStruct((B, ROW_O), jnp.float32),
        mesh=mesh,
        compiler_params=pltpu.CompilerParams(
            needs_layout_passes=False, use_tc_tiling_on_sc=False),
        scratch_types=[
            pltpu.VMEM((HALF,), jnp.int32),
            pltpu.VMEM((HALF + 32,), jnp.int32),
            pltpu.VMEM((ROW_W,), jnp.float32),
            pltpu.VMEM((G, ROW_X), jnp.float32),
            pltpu.VMEM((G, ROW_O), jnp.float32),
            pltpu.SemaphoreType.DMA,
            pltpu.SemaphoreType.DMA,
        ],
    )(x_flat, bid, w_flat)


def kernel(input, beam_id, W):
    x_flat = input.reshape(B, ROW_X)
    w_flat = jnp.transpose(W, (0, 2, 4, 1, 3)).reshape(NBEAMS, ROW_W)
    bid = beam_id.astype(jnp.int32)
    out = _beamform_sc(x_flat, bid, w_flat)
    return out.reshape(B, 2, F, N)
